# baseline (speedup 1.0000x reference)
import jax
import jax.numpy as jnp
from jax.experimental import pallas as pl

C = 128
NUMD = 400000
N_PARENT = 80000
LEAF_NUM = 30000
PREFIX = 20000
TOTAL_OUT = PREFIX + N_PARENT

BP = 3200
NPRE = 7
NPAR = N_PARENT // BP
LEAF_B = 3 * BP // 8
CHILD_B = 5 * BP
MM_B = 5 * BP // 8
GRP = BP // 8
PRE_LAST = PREFIX - BP
B8 = BP // 8


def _fused_kernel(pref_ref, leaf_ref, child_ref, w_ref, out_ref):
    i = pl.program_id(0)

    @pl.when(i < NPRE)
    def _prefix_copy():
        out_ref[...] = pref_ref[...]

    @pl.when(i >= NPRE)
    def _parent_block():
        leaf = leaf_ref[...]
        sub_rows = 8 * (CHILD_BASE8 + (CHILD_B // 8) * (i - NPRE)
                        - _child_woff8(i))
        xd = child_ref[pl.ds(sub_rows, CHILD_B), :].reshape(MM_B, 8 * C)
        outd = jnp.dot(xd, w_ref[...], preferred_element_type=jnp.float32)
        merged = jnp.concatenate(
            [leaf.reshape(GRP, 3, C), outd.reshape(GRP, 5, C)], axis=1)
        out_ref[...] = merged.reshape(BP, C)


def _pref_off(i):
    return 8 * jnp.minimum((BP // 8) * i, PRE_LAST // 8)


def _out_off(i):
    return 8 * jnp.where(i < NPRE,
                         jnp.minimum(B8 * i, PRE_LAST // 8),
                         PREFIX // 8 + B8 * (i - NPRE))


def _leaf_off(i):
    return 8 * (PREFIX // 8 + (LEAF_B // 8) * jnp.maximum(i - NPRE, 0))


CHILD_BASE8 = (PREFIX + LEAF_NUM) // 8
CHILD_WMAX8 = (PREFIX + LEAF_NUM + NUMD - 2 * CHILD_B) // 8


def _child_woff8(i):
    return jnp.minimum(
        CHILD_BASE8 + (CHILD_B // 4) * (jnp.maximum(i - NPRE, 0) // 2),
        CHILD_WMAX8)


def _child_off(i):
    return 8 * _child_woff8(i)


def kernel(x, children, W):
    del children
    weights = W.reshape(C, C * 8).T
    return pl.pallas_call(
        _fused_kernel,
        grid=(NPRE + NPAR,),
        in_specs=[
            pl.BlockSpec((pl.Element(BP), pl.Element(C)),
                         lambda i: (_pref_off(i), 0)),
            pl.BlockSpec((pl.Element(LEAF_B), pl.Element(C)),
                         lambda i: (_leaf_off(i), 0)),
            pl.BlockSpec((pl.Element(2 * CHILD_B), pl.Element(C)),
                         lambda i: (_child_off(i), 0)),
            pl.BlockSpec((C * 8, C), lambda i: (0, 0)),
        ],
        out_specs=pl.BlockSpec((pl.Element(BP), pl.Element(C)),
                               lambda i: (_out_off(i), 0)),
        out_shape=jax.ShapeDtypeStruct((TOTAL_OUT, C), x.dtype),
    )(x, x, x, weights)


# speedup vs baseline: 1.5449x; 1.5449x over previous
import jax
import jax.numpy as jnp
from jax.experimental import pallas as pl
from jax.experimental.pallas import tpu as pltpu

C = 128
NUMD = 400000
N_PARENT = 80000
LEAF_NUM = 30000
PREFIX = 20000
TOTAL_OUT = PREFIX + N_PARENT

BP = 3200
NPRE = 7
NPAR = N_PARENT // BP
LEAF_B = 3 * BP // 8
MM_B = 5 * BP // 8
GRP = BP // 8
PRE_LAST = PREFIX - BP
B8 = BP // 8
CHILD_G0 = (PREFIX + LEAF_NUM) // 8
NSLOT = 3


def _fused_kernel(xv_ref, pref_ref, leaf_ref, w_ref, out_ref,
                  child_buf, sem):
    i = pl.program_id(0)

    jf = i - (NPRE - 2)

    @pl.when(jnp.logical_and(jf >= 0, jf < NPAR))
    def _start_fetch():
        slot = jax.lax.rem(jf, NSLOT)
        g0 = CHILD_G0 + MM_B * jf
        for t in range(8):
            pltpu.make_async_copy(
                xv_ref.at[pl.ds(g0, MM_B), t, :],
                child_buf.at[slot, t],
                sem.at[slot, t],
            ).start()

    @pl.when(i < NPRE)
    def _prefix_copy():
        out_ref[...] = pref_ref[...]

    @pl.when(i >= NPRE)
    def _parent_block():
        j = i - NPRE
        slot = jax.lax.rem(j, NSLOT)
        for t in range(8):
            pltpu.make_async_copy(
                xv_ref.at[pl.ds(CHILD_G0 + MM_B * j, MM_B), t, :],
                child_buf.at[slot, t],
                sem.at[slot, t],
            ).wait()
        outd = jnp.dot(child_buf[slot, 0], w_ref[0],
                       preferred_element_type=jnp.float32)
        for t in range(1, 8):
            outd = outd + jnp.dot(child_buf[slot, t], w_ref[t],
                                  preferred_element_type=jnp.float32)
        merged = jnp.concatenate(
            [leaf_ref[...].reshape(GRP, 3, C), outd.reshape(GRP, 5, C)],
            axis=1)
        out_ref[...] = merged.reshape(BP, C)


def _pref_off(i):
    return 8 * jnp.minimum(B8 * i, PRE_LAST // 8)


def _out_off(i):
    return 8 * jnp.where(i < NPRE,
                         jnp.minimum(B8 * i, PRE_LAST // 8),
                         PREFIX // 8 + B8 * (i - NPRE))


def _leaf_off(i):
    return 8 * (PREFIX // 8 + (LEAF_B // 8) * jnp.maximum(i - NPRE, 0))


def kernel(x, children, W):
    del children
    xv = x.reshape(NUMD // 8 + CHILD_G0, 8, C)
    weights = W.reshape(C, C * 8).T.reshape(8, C, C)
    return pl.pallas_call(
        _fused_kernel,
        grid=(NPRE + NPAR,),
        in_specs=[
            pl.BlockSpec(memory_space=pl.ANY),
            pl.BlockSpec((pl.Element(BP), pl.Element(C)),
                         lambda i: (_pref_off(i), 0)),
            pl.BlockSpec((pl.Element(LEAF_B), pl.Element(C)),
                         lambda i: (_leaf_off(i), 0)),
            pl.BlockSpec((8, C, C), lambda i: (0, 0, 0)),
        ],
        out_specs=pl.BlockSpec((pl.Element(BP), pl.Element(C)),
                               lambda i: (_out_off(i), 0)),
        out_shape=jax.ShapeDtypeStruct((TOTAL_OUT, C), x.dtype),
        scratch_shapes=[
            pltpu.VMEM((NSLOT, 8, MM_B, C), jnp.float32),
            pltpu.SemaphoreType.DMA((NSLOT, 8)),
        ],
    )(xv, x, x, weights)
